# Initial kernel scaffold; baseline (speedup 1.0000x reference)
import jax
import jax.numpy as jnp
from jax.experimental import pallas as pl


def kernel(x, edge_index, pos_encoding, params):
    raise NotImplementedError("write your pallas kernel here")


# speedup vs baseline: 4.4167x; 4.4167x over previous
import functools

import jax
import jax.numpy as jnp
from jax import lax
from jax.experimental import pallas as pl
from jax.experimental.pallas import tpu as pltpu
from jax.experimental.pallas import tpu_sc as plsc

RB = 1000


def _mm_bias_body(x_ref, w_ref, b_ref, o_ref):
    o_ref[...] = (
        jnp.dot(x_ref[...], w_ref[...], preferred_element_type=jnp.float32)
        + b_ref[...]
    )


def _tc_mm_bias(x, w, b):
    n, din = x.shape
    h = w.shape[1]
    return pl.pallas_call(
        _mm_bias_body,
        grid=(n // RB,),
        in_specs=[
            pl.BlockSpec((RB, din), lambda i: (i, 0)),
            pl.BlockSpec((din, h), lambda i: (0, 0)),
            pl.BlockSpec((1, h), lambda i: (0, 0)),
        ],
        out_specs=pl.BlockSpec((RB, h), lambda i: (i, 0)),
        out_shape=jax.ShapeDtypeStruct((n, h), jnp.float32),
    )(x, w, b.reshape(1, h))


def _stage_a_body(h_ref, pe_ref, wc_ref, ws_ref, bk_ref, wpe_ref, hin_ref, cs_ref):
    hb = h_ref[...]
    hin = (
        jnp.dot(jnp.cos(hb), wc_ref[...], preferred_element_type=jnp.float32)
        + jnp.dot(jnp.sin(hb), ws_ref[...], preferred_element_type=jnp.float32)
        + bk_ref[...]
        + jnp.dot(pe_ref[...], wpe_ref[...], preferred_element_type=jnp.float32)
    )
    hin_ref[...] = hin

    @pl.when(pl.program_id(0) == 0)
    def _():
        cs_ref[...] = jnp.zeros_like(cs_ref)

    cs_ref[...] += jnp.sum(hin, axis=0, keepdims=True)


def _tc_stage_a(h, pe, wc, ws, bk, wpe):
    n, hd = h.shape
    p = pe.shape[1]
    return pl.pallas_call(
        _stage_a_body,
        grid=(n // RB,),
        in_specs=[
            pl.BlockSpec((RB, hd), lambda i: (i, 0)),
            pl.BlockSpec((RB, p), lambda i: (i, 0)),
            pl.BlockSpec((hd, hd), lambda i: (0, 0)),
            pl.BlockSpec((hd, hd), lambda i: (0, 0)),
            pl.BlockSpec((1, hd), lambda i: (0, 0)),
            pl.BlockSpec((p, hd), lambda i: (0, 0)),
        ],
        out_specs=[
            pl.BlockSpec((RB, hd), lambda i: (i, 0)),
            pl.BlockSpec((1, hd), lambda i: (0, 0)),
        ],
        out_shape=[
            jax.ShapeDtypeStruct((n, hd), jnp.float32),
            jax.ShapeDtypeStruct((1, hd), jnp.float32),
        ],
    )(h, pe, wc, ws, bk.reshape(1, hd), wpe)


def _stage_b_body(inv_n, hin_ref, a0_ref, a1_ref, d0_ref, d1_ref, cs_ref,
                  wl_ref, bl_ref, wg_ref, o_ref):
    deg = jnp.maximum(d0_ref[...] + d1_ref[...], 1.0)
    agg = (a0_ref[...] + a1_ref[...]) / deg
    local = (
        jnp.dot(agg, wl_ref[...], preferred_element_type=jnp.float32)
        + bl_ref[...]
    )
    glob = jnp.dot(cs_ref[...] * inv_n, wg_ref[...],
                   preferred_element_type=jnp.float32)
    o_ref[...] = jnp.maximum(hin_ref[...] + local + glob, 0.0)


def _tc_stage_b(hin, a0, a1, d0, d1, cs, wl, bl, wg):
    n, hd = hin.shape
    return pl.pallas_call(
        functools.partial(_stage_b_body, 1.0 / n),
        grid=(n // RB,),
        in_specs=[
            pl.BlockSpec((RB, hd), lambda i: (i, 0)),
            pl.BlockSpec((RB, hd), lambda i: (i, 0)),
            pl.BlockSpec((RB, hd), lambda i: (i, 0)),
            pl.BlockSpec((RB, 1), lambda i: (i, 0)),
            pl.BlockSpec((RB, 1), lambda i: (i, 0)),
            pl.BlockSpec((1, hd), lambda i: (0, 0)),
            pl.BlockSpec((hd, hd), lambda i: (0, 0)),
            pl.BlockSpec((1, hd), lambda i: (0, 0)),
            pl.BlockSpec((hd, hd), lambda i: (0, 0)),
        ],
        out_specs=pl.BlockSpec((RB, hd), lambda i: (i, 0)),
        out_shape=jax.ShapeDtypeStruct((n, hd), jnp.float32),
    )(hin, a0, a1, d0, d1, cs, wl, bl.reshape(1, hd), wg)


def _chunk_size(per_tile):
    for c in range(128, 0, -8):
        if per_tile % c == 0:
            return c
    return 8


def _make_sc_agg(n, hd, e, nc, ns):
    nw = nc * ns
    per_tile = e // nw
    chunk = _chunk_size(per_tile)
    n_iters = per_tile // chunk
    zb_step = (n // ns) // 8 * 8
    zb_len = n - (ns - 1) * zb_step
    mesh = plsc.VectorSubcoreMesh(core_axis_name="c", subcore_axis_name="s")

    @functools.partial(
        pl.kernel,
        mesh=mesh,
        out_type=jax.ShapeDtypeStruct((nc, n, hd), jnp.float32),
        scratch_types=[
            pltpu.VMEM((chunk,), jnp.int32),
            pltpu.VMEM((chunk,), jnp.int32),
            pltpu.VMEM((chunk, hd), jnp.float32),
            pltpu.SemaphoreType.DMA,
            pltpu.VMEM_SHARED((n, hd), jnp.float32),
        ],
    )
    def k(h_hbm, src_hbm, dst_hbm, z_hbm, out_hbm, sidx, didx, rows, sem, acc):
        c = lax.axis_index("c")
        s = lax.axis_index("s")
        wid = c * ns + s
        zb = s * zb_step
        pltpu.sync_copy(z_hbm.at[pl.ds(zb, zb_len)], acc.at[pl.ds(zb, zb_len)])
        plsc.subcore_barrier()

        def body(j, carry):
            base = wid * per_tile + j * chunk
            pltpu.sync_copy(src_hbm.at[pl.ds(base, chunk)], sidx)
            pltpu.sync_copy(dst_hbm.at[pl.ds(base, chunk)], didx)
            pltpu.async_copy(h_hbm.at[sidx], rows, sem).wait()
            pltpu.sync_copy(rows, acc.at[didx], add=True)
            return carry

        lax.fori_loop(0, n_iters, body, 0)
        plsc.subcore_barrier()
        pltpu.sync_copy(acc.at[pl.ds(zb, zb_len)],
                        out_hbm.at[c, pl.ds(zb, zb_len)])

    return k


def _make_sc_deg(n, e, nc, ns):
    nw = nc * ns
    per_tile = e // nw
    chunk = _chunk_size(per_tile)
    n_iters = per_tile // chunk
    zb_step = (n // ns) // 8 * 8
    zb_len = n - (ns - 1) * zb_step
    mesh = plsc.VectorSubcoreMesh(core_axis_name="c", subcore_axis_name="s")

    @functools.partial(
        pl.kernel,
        mesh=mesh,
        out_type=jax.ShapeDtypeStruct((nc, n, 128), jnp.float32),
        scratch_types=[
            pltpu.VMEM((chunk,), jnp.int32),
            pltpu.VMEM((chunk, 128), jnp.float32),
            pltpu.VMEM_SHARED((n, 128), jnp.float32),
        ],
    )
    def k(dst_hbm, z_hbm, ones_hbm, out_hbm, didx, ones, acc):
        c = lax.axis_index("c")
        s = lax.axis_index("s")
        wid = c * ns + s
        pltpu.sync_copy(ones_hbm, ones)
        zb = s * zb_step
        pltpu.sync_copy(z_hbm.at[pl.ds(zb, zb_len)], acc.at[pl.ds(zb, zb_len)])
        plsc.subcore_barrier()

        def body(j, carry):
            base = wid * per_tile + j * chunk
            pltpu.sync_copy(dst_hbm.at[pl.ds(base, chunk)], didx)
            pltpu.sync_copy(ones, acc.at[didx], add=True)
            return carry

        lax.fori_loop(0, n_iters, body, 0)
        plsc.subcore_barrier()
        pltpu.sync_copy(acc.at[pl.ds(zb, zb_len)],
                        out_hbm.at[c, pl.ds(zb, zb_len)])

    return k


def kernel(x, edge_index, pos_encoding, params):
    n, _ = x.shape
    hd = params['W0'].shape[1]
    e = edge_index.shape[1]
    info = plsc.get_sparse_core_info()
    nc, ns = info.num_cores, info.num_subcores

    src = edge_index[0].astype(jnp.int32)
    dst = edge_index[1].astype(jnp.int32)
    zeros2 = jnp.zeros((n, hd), jnp.float32)
    chunk = _chunk_size(e // (nc * ns))
    ones_c = jnp.ones((chunk, 128), jnp.float32)

    deg_p = _make_sc_deg(n, e, nc, ns)(dst, zeros2, ones_c)
    d0 = deg_p[0, :, 0].reshape(n, 1)
    d1 = deg_p[1, :, 0].reshape(n, 1)

    h = _tc_mm_bias(x, params['W0'], params['b0'])
    agg_fn = _make_sc_agg(n, hd, e, nc, ns)
    for p in params['layers']:
        hin, cs = _tc_stage_a(h, pos_encoding, p['Wc'], p['Ws'], p['bk'],
                              p['Wpe'])
        agg_p = agg_fn(hin, src, dst, zeros2)
        h = _tc_stage_b(hin, agg_p[0], agg_p[1], d0, d1, cs, p['Wl'],
                        p['bl'], p['Wg'])
    return _tc_mm_bias(h, params['Wf'], params['bf'])
